# baseline (device time: 35479 ns/iter reference)
import jax
import jax.numpy as jnp
from jax import lax
from jax.experimental import pallas as pl
from jax.experimental.pallas import tpu as pltpu

N_LAYERS = 3
CY = 4
CX = 2


def kernel(x, Win0, Wout0, Win1, Wout1, Win2, Wout2):
    b, d_loc = x.shape
    _, h_loc = Win0.shape

    def body(x_ref, win0_ref, wout0_ref, win1_ref, wout1_ref, win2_ref,
             wout2_ref, out_ref, send_h, recv_h, send_x, recv_x,
             send_sems, recv_sems):
        my_x = lax.axis_index("x")
        my_y = lax.axis_index("y")
        y_peer = (my_x, 1 - my_y)
        x_peer = (1 - my_x, my_y)

        barrier = pltpu.get_barrier_semaphore()
        for peer in (y_peer, x_peer):
            pl.semaphore_signal(
                barrier, inc=1,
                device_id=peer, device_id_type=pl.DeviceIdType.MESH,
            )
        pl.semaphore_wait(barrier, 2)

        win_refs = [win0_ref, win1_ref, win2_ref]
        wout_refs = [wout0_ref, wout1_ref, wout2_ref]

        wy = h_loc // CY
        wx = d_loc // CX

        ph = jnp.dot(
            x_ref[:, :], win_refs[0][:, :], preferred_element_type=jnp.float32
        )

        for layer in range(N_LAYERS):
            e_h = 2 * layer
            e_x = 2 * layer + 1

            rdmas_h = []
            for c in range(CY):
                cs = slice(c * wy, (c + 1) * wy)
                send_h[:, cs] = ph[:, cs].astype(jnp.bfloat16)
                r = pltpu.make_async_remote_copy(
                    src_ref=send_h.at[:, cs],
                    dst_ref=recv_h.at[layer, :, cs],
                    send_sem=send_sems.at[e_h, c],
                    recv_sem=recv_sems.at[e_h, c],
                    device_id=y_peer,
                    device_id_type=pl.DeviceIdType.MESH,
                )
                r.start()
                rdmas_h.append(r)
            px = jnp.zeros((b, d_loc), jnp.float32)
            for c in range(CY):
                cs = slice(c * wy, (c + 1) * wy)
                rdmas_h[c].wait()
                h_c = jnp.maximum(
                    ph[:, cs] + recv_h[layer, :, cs].astype(jnp.float32), 0.0
                )
                px = px + jnp.dot(
                    h_c, wout_refs[layer][cs, :],
                    preferred_element_type=jnp.float32,
                )

            rdmas_x = []
            for k in range(CX):
                ks = slice(k * wx, (k + 1) * wx)
                send_x[:, ks] = px[:, ks].astype(jnp.bfloat16)
                r = pltpu.make_async_remote_copy(
                    src_ref=send_x.at[:, ks],
                    dst_ref=recv_x.at[layer, :, ks],
                    send_sem=send_sems.at[e_x, k],
                    recv_sem=recv_sems.at[e_x, k],
                    device_id=x_peer,
                    device_id_type=pl.DeviceIdType.MESH,
                )
                r.start()
                rdmas_x.append(r)
            if layer + 1 < N_LAYERS:
                ph = jnp.zeros((b, h_loc), jnp.float32)
            for k in range(CX):
                ks = slice(k * wx, (k + 1) * wx)
                rdmas_x[k].wait()
                xk = px[:, ks] + recv_x[layer, :, ks].astype(jnp.float32)
                if layer + 1 < N_LAYERS:
                    ph = ph + jnp.dot(
                        xk, win_refs[layer + 1][ks, :],
                        preferred_element_type=jnp.float32,
                    )
                else:
                    out_ref[:, ks] = xk

    return pl.pallas_call(
        body,
        out_shape=jax.ShapeDtypeStruct((b, d_loc), jnp.float32),
        in_specs=[pl.BlockSpec(memory_space=pltpu.VMEM)] * 7,
        out_specs=pl.BlockSpec(memory_space=pltpu.VMEM),
        scratch_shapes=[
            pltpu.VMEM((b, h_loc), jnp.bfloat16),
            pltpu.VMEM((N_LAYERS, b, h_loc), jnp.bfloat16),
            pltpu.VMEM((b, d_loc), jnp.bfloat16),
            pltpu.VMEM((N_LAYERS, b, d_loc), jnp.bfloat16),
            pltpu.SemaphoreType.DMA((2 * N_LAYERS, CY)),
            pltpu.SemaphoreType.DMA((2 * N_LAYERS, CY)),
        ],
        compiler_params=pltpu.CompilerParams(collective_id=0),
    )(x, Win0, Wout0, Win1, Wout1, Win2, Wout2)


# device time: 28430 ns/iter; 1.2479x vs baseline; 1.2479x over previous
import jax
import jax.numpy as jnp
from jax import lax
from jax.experimental import pallas as pl
from jax.experimental.pallas import tpu as pltpu

N_LAYERS = 3
MB = 4
CY = 1
CX = 1


def kernel(x, Win0, Wout0, Win1, Wout1, Win2, Wout2):
    b, d_loc = x.shape
    _, h_loc = Win0.shape
    mb_rows = b // MB
    wy = h_loc // CY
    wx = d_loc // CX

    def body(x_ref, win0_ref, wout0_ref, win1_ref, wout1_ref, win2_ref,
             wout2_ref, out_ref, send_h, recv_h, send_x, recv_x,
             winb, woutb, send_sems, recv_sems):
        my_x = lax.axis_index("x")
        my_y = lax.axis_index("y")
        y_peer = (my_x, 1 - my_y)
        x_peer = (1 - my_x, my_y)

        win_refs = [win0_ref, win1_ref, win2_ref]
        wout_refs = [wout0_ref, wout1_ref, wout2_ref]

        def rows(m):
            return slice(m * mb_rows, (m + 1) * mb_rows)

        def start_y(m, layer, ph, store=True):
            e = 2 * (N_LAYERS * m + layer)
            rdmas = []
            for c in range(CY):
                cs = slice(c * wy, (c + 1) * wy)
                if store:
                    send_h[rows(m), cs] = ph[:, cs].astype(jnp.bfloat16)
                r = pltpu.make_async_remote_copy(
                    src_ref=send_h.at[rows(m), cs],
                    dst_ref=recv_h.at[layer, rows(m), cs],
                    send_sem=send_sems.at[e, c],
                    recv_sem=recv_sems.at[e, c],
                    device_id=y_peer,
                    device_id_type=pl.DeviceIdType.MESH,
                )
                r.start()
                rdmas.append(r)
            return rdmas

        def consume_y(m, layer, ph, rdmas):
            px = jnp.zeros((mb_rows, d_loc), jnp.float32)
            for c in range(CY):
                cs = slice(c * wy, (c + 1) * wy)
                rdmas[c].wait()
                h_c = jnp.maximum(
                    ph[:, cs] + recv_h[layer, rows(m), cs].astype(jnp.float32),
                    0.0,
                )
                px = px + jnp.dot(
                    h_c.astype(jnp.bfloat16), woutb[layer, cs, :],
                    preferred_element_type=jnp.float32,
                )
            return px

        def start_x(m, layer, px):
            e = 2 * (N_LAYERS * m + layer) + 1
            rdmas = []
            for k in range(CX):
                ks = slice(k * wx, (k + 1) * wx)
                send_x[rows(m), ks] = px[:, ks].astype(jnp.bfloat16)
                r = pltpu.make_async_remote_copy(
                    src_ref=send_x.at[rows(m), ks],
                    dst_ref=recv_x.at[layer, rows(m), ks],
                    send_sem=send_sems.at[e, k],
                    recv_sem=recv_sems.at[e, k],
                    device_id=x_peer,
                    device_id_type=pl.DeviceIdType.MESH,
                )
                r.start()
                rdmas.append(r)
            return rdmas

        def consume_x(m, layer, px, rdmas):
            last = layer + 1 == N_LAYERS
            ph_next = None if last else jnp.zeros((mb_rows, h_loc), jnp.float32)
            for k in range(CX):
                ks = slice(k * wx, (k + 1) * wx)
                rdmas[k].wait()
                xk = px[:, ks] + recv_x[layer, rows(m), ks].astype(jnp.float32)
                if last:
                    out_ref[rows(m), ks] = xk
                else:
                    ph_next = ph_next + jnp.dot(
                        xk.astype(jnp.bfloat16), winb[layer + 1, ks, :],
                        preferred_element_type=jnp.float32,
                    )
            return ph_next

        ph = [None] * MB
        px = [None] * MB
        ry = [None] * MB
        rx = [None] * MB

        winb[0, :, :] = win_refs[0][:, :].astype(jnp.bfloat16)
        for m in range(MB):
            ph[m] = jnp.dot(
                x_ref[rows(m), :].astype(jnp.bfloat16), winb[0, :, :],
                preferred_element_type=jnp.float32,
            )
            for c in range(CY):
                cs = slice(c * wy, (c + 1) * wy)
                send_h[rows(m), cs] = ph[m][:, cs].astype(jnp.bfloat16)

        barrier = pltpu.get_barrier_semaphore()
        for peer in (y_peer, x_peer):
            pl.semaphore_signal(
                barrier, inc=1,
                device_id=peer, device_id_type=pl.DeviceIdType.MESH,
            )
        pl.semaphore_wait(barrier, 2)

        for m in range(MB):
            ry[m] = start_y(m, 0, ph[m], store=False)

        for i in range(N_LAYERS):
            if i > 0:
                winb[i, :, :] = win_refs[i][:, :].astype(jnp.bfloat16)
            woutb[i, :, :] = wout_refs[i][:, :].astype(jnp.bfloat16)

        for layer in range(N_LAYERS):
            for m in range(MB):
                px[m] = consume_y(m, layer, ph[m], ry[m])
                rx[m] = start_x(m, layer, px[m])
            for m in range(MB):
                ph[m] = consume_x(m, layer, px[m], rx[m])
                if layer + 1 < N_LAYERS:
                    ry[m] = start_y(m, layer + 1, ph[m])

    n_exch = 2 * N_LAYERS * MB
    return pl.pallas_call(
        body,
        out_shape=jax.ShapeDtypeStruct((b, d_loc), jnp.float32),
        in_specs=[pl.BlockSpec(memory_space=pltpu.VMEM)] * 7,
        out_specs=pl.BlockSpec(memory_space=pltpu.VMEM),
        scratch_shapes=[
            pltpu.VMEM((b, h_loc), jnp.bfloat16),
            pltpu.VMEM((N_LAYERS, b, h_loc), jnp.bfloat16),
            pltpu.VMEM((b, d_loc), jnp.bfloat16),
            pltpu.VMEM((N_LAYERS, b, d_loc), jnp.bfloat16),
            pltpu.VMEM((N_LAYERS, d_loc, h_loc), jnp.bfloat16),
            pltpu.VMEM((N_LAYERS, h_loc, d_loc), jnp.bfloat16),
            pltpu.SemaphoreType.DMA((n_exch, CY)),
            pltpu.SemaphoreType.DMA((n_exch, CY)),
        ],
        compiler_params=pltpu.CompilerParams(collective_id=0),
    )(x, Win0, Wout0, Win1, Wout1, Win2, Wout2)
